# baseline (speedup 1.0000x reference)
import jax
import jax.numpy as jnp
from jax.experimental import pallas as pl

N = 10000
POOL_RATIO = 0.5
BM = BN = BK = 512


def _mm_kernel(a_ref, b_ref, o_ref):
    @pl.when(pl.program_id(2) == 0)
    def _init():
        o_ref[...] = jnp.zeros_like(o_ref)

    o_ref[...] += jnp.dot(a_ref[...], b_ref[...],
                          preferred_element_type=jnp.float32)


def _mm_mask_kernel(a_ref, b_ref, o_ref):
    _mm_kernel(a_ref, b_ref, o_ref)

    @pl.when(pl.program_id(2) == pl.num_programs(2) - 1)
    def _mask():
        shp = o_ref.shape
        r = pl.program_id(0) * shp[0] + jax.lax.broadcasted_iota(
            jnp.int32, shp, 0)
        c = pl.program_id(1) * shp[1] + jax.lax.broadcasted_iota(
            jnp.int32, shp, 1)
        o_ref[...] = jnp.where(r == c, 0.0, o_ref[...])


def _mm(a, b, mask_diag=False):
    m, k = a.shape
    _, n = b.shape
    bn = min(BN, n)
    return pl.pallas_call(
        _mm_mask_kernel if mask_diag else _mm_kernel,
        grid=(m // BM, n // bn, k // BK),
        in_specs=[
            pl.BlockSpec((BM, BK), lambda i, j, kk: (i, kk)),
            pl.BlockSpec((BK, bn), lambda i, j, kk: (kk, j)),
        ],
        out_specs=pl.BlockSpec((BM, bn), lambda i, j, kk: (i, j)),
        out_shape=jax.ShapeDtypeStruct((m, n), jnp.float32),
    )(a, b)


def _logsoftmax_kernel(z_ref, o_ref):
    z = z_ref[...]
    m = jnp.max(z, axis=1, keepdims=True)
    e = jnp.exp(z - m)
    o_ref[...] = (z - m) - jnp.log(jnp.sum(e, axis=1, keepdims=True))


def _gcn(xw, a_pad, deg, b):
    n, h = xw.shape
    dinv = jax.lax.rsqrt(deg)
    z = dinv[:, None] * xw
    zp = jnp.pad(z, ((0, a_pad.shape[1] - n), (0, 128 - h)))
    az = _mm(a_pad, zp)[:n, :h]
    return dinv[:, None] * (az + 2.0 * z) + b


def kernel(x, edge_index, W0, b0, p1, W1, b1, p2, W2, b2, Wu0, bu0, Wu1, bu1):
    n1 = x.shape[0]
    n2 = (n1 + 1) // 2
    n3 = (n2 + 1) // 2
    P1, P2, P3 = 10240, 5120, 2560
    src, dst = edge_index[0], edge_index[1]

    a1 = jnp.zeros((P1, P1), jnp.float32).at[dst, src].add(1.0)
    deg1 = jnp.zeros((n1,), jnp.float32).at[dst].add(1.0) + 2.0

    x1 = jax.nn.relu(_gcn(x @ W0, a1, deg1, b0))

    score = jnp.tanh((x1 @ p1) / jnp.linalg.norm(p1))
    vals, perm1 = jax.lax.top_k(score, n2)
    xp = x1[perm1] * vals[:, None]
    ar2 = jnp.arange(n2, dtype=jnp.int32)
    inv1 = jnp.full((n1,), P2 + 7, jnp.int32).at[perm1].set(ar2)
    rows = (jnp.zeros((P2, P1), jnp.float32)
            .at[inv1[dst], src].add(1.0, mode="drop")
            .at[ar2, perm1].add(1.0))
    cols = (jnp.zeros((P1, P2), jnp.float32)
            .at[dst, inv1[src]].add(1.0, mode="drop")
            .at[perm1, ar2].add(1.0))
    a2 = _mm(rows.astype(jnp.bfloat16), cols.astype(jnp.bfloat16),
             mask_diag=True)
    deg2 = jnp.sum(a2, axis=1)[:n2] + 2.0

    x2 = jax.nn.relu(_gcn(xp @ W1, a2, deg2, b1))

    score2 = jnp.tanh((x2 @ p2) / jnp.linalg.norm(p2))
    vals2, perm2 = jax.lax.top_k(score2, n3)
    xp2 = x2[perm2] * vals2[:, None]
    ar3 = jnp.arange(n3, dtype=jnp.int32)
    a2d = a2[:n2, :n2]
    rows2 = (jnp.pad(a2d[perm2, :], ((0, P3 - n3), (0, P2 - n2)))
             .at[ar3, perm2].add(1.0))
    cols2 = (jnp.pad(a2d[:, perm2], ((0, P2 - n2), (0, P3 - n3)))
             .at[perm2, ar3].add(1.0))
    a3 = _mm(rows2.astype(jnp.bfloat16), cols2.astype(jnp.bfloat16),
             mask_diag=True)
    deg3 = jnp.sum(a3, axis=1)[:n3] + 2.0

    x3 = jax.nn.relu(_gcn(xp2 @ W2, a3, deg3, b2))

    up = jnp.zeros_like(x2).at[perm2].set(x3)
    xu = jax.nn.relu(_gcn(jnp.concatenate([x2, up], axis=1) @ Wu0,
                          a2, deg2, bu0))
    up1 = jnp.zeros_like(x1).at[perm1].set(xu)
    out = _gcn(jnp.concatenate([x1, up1], axis=1) @ Wu1, a1, deg1, bu1)

    return pl.pallas_call(
        _logsoftmax_kernel,
        out_shape=jax.ShapeDtypeStruct(out.shape, out.dtype),
    )(out)


# speedup vs baseline: 1.0871x; 1.0871x over previous
import jax
import jax.numpy as jnp
from jax.experimental import pallas as pl

N = 10000
POOL_RATIO = 0.5
BM = BN = BK = 512


def _mm_kernel(a_ref, b_ref, o_ref):
    @pl.when(pl.program_id(2) == 0)
    def _init():
        o_ref[...] = jnp.zeros_like(o_ref)

    o_ref[...] += jnp.dot(a_ref[...], b_ref[...],
                          preferred_element_type=jnp.float32)


def _mm_mask_kernel(a_ref, b_ref, o_ref):
    _mm_kernel(a_ref, b_ref, o_ref)

    @pl.when(pl.program_id(2) == pl.num_programs(2) - 1)
    def _mask():
        shp = o_ref.shape
        r = pl.program_id(0) * shp[0] + jax.lax.broadcasted_iota(
            jnp.int32, shp, 0)
        c = pl.program_id(1) * shp[1] + jax.lax.broadcasted_iota(
            jnp.int32, shp, 1)
        o_ref[...] = jnp.where(r == c, 0.0, o_ref[...])


def _mm(a, b, mask_diag=False):
    m, k = a.shape
    _, n = b.shape
    bn = min(BN, n)
    return pl.pallas_call(
        _mm_mask_kernel if mask_diag else _mm_kernel,
        grid=(m // BM, n // bn, k // BK),
        in_specs=[
            pl.BlockSpec((BM, BK), lambda i, j, kk: (i, kk)),
            pl.BlockSpec((BK, bn), lambda i, j, kk: (kk, j)),
        ],
        out_specs=pl.BlockSpec((BM, bn), lambda i, j, kk: (i, j)),
        out_shape=jax.ShapeDtypeStruct((m, n), jnp.float32),
    )(a, b)


def _mmt_mask_kernel(a_ref, bt_ref, o_ref):
    @pl.when(pl.program_id(2) == 0)
    def _init():
        o_ref[...] = jnp.zeros_like(o_ref)

    o_ref[...] += jax.lax.dot_general(
        a_ref[...], bt_ref[...], (((1,), (1,)), ((), ())),
        preferred_element_type=jnp.float32)

    @pl.when(pl.program_id(2) == pl.num_programs(2) - 1)
    def _mask():
        shp = o_ref.shape
        r = pl.program_id(0) * shp[0] + jax.lax.broadcasted_iota(
            jnp.int32, shp, 0)
        c = pl.program_id(1) * shp[1] + jax.lax.broadcasted_iota(
            jnp.int32, shp, 1)
        o_ref[...] = jnp.where(r == c, 0.0, o_ref[...])


def _mmt_mask(a, bt):
    m, k = a.shape
    n = bt.shape[0]
    return pl.pallas_call(
        _mmt_mask_kernel,
        grid=(m // BM, n // BN, k // BK),
        in_specs=[
            pl.BlockSpec((BM, BK), lambda i, j, kk: (i, kk)),
            pl.BlockSpec((BN, BK), lambda i, j, kk: (j, kk)),
        ],
        out_specs=pl.BlockSpec((BM, BN), lambda i, j, kk: (i, j)),
        out_shape=jax.ShapeDtypeStruct((m, n), jnp.float32),
    )(a, bt)


def _logsoftmax_kernel(z_ref, o_ref):
    z = z_ref[...]
    m = jnp.max(z, axis=1, keepdims=True)
    e = jnp.exp(z - m)
    o_ref[...] = (z - m) - jnp.log(jnp.sum(e, axis=1, keepdims=True))


def _gcn(xw, a_pad, deg, b):
    n, h = xw.shape
    dinv = jax.lax.rsqrt(deg)
    z = dinv[:, None] * xw
    zp = jnp.pad(z, ((0, a_pad.shape[1] - n), (0, 128 - h)))
    az = _mm(a_pad, zp)[:n, :h]
    return dinv[:, None] * (az + 2.0 * z) + b


def kernel(x, edge_index, W0, b0, p1, W1, b1, p2, W2, b2, Wu0, bu0, Wu1, bu1):
    n1 = x.shape[0]
    n2 = (n1 + 1) // 2
    n3 = (n2 + 1) // 2
    P1, P2, P3 = 10240, 5120, 2560
    src, dst = edge_index[0], edge_index[1]

    a1 = jnp.zeros((P1, P1), jnp.float32).at[dst, src].add(1.0)
    a1t = jnp.zeros((P1, P1), jnp.float32).at[src, dst].add(1.0)
    deg1 = jnp.zeros((n1,), jnp.float32).at[dst].add(1.0) + 2.0

    x1 = jax.nn.relu(_gcn(x @ W0, a1, deg1, b0))

    score = jnp.tanh((x1 @ p1) / jnp.linalg.norm(p1))
    vals, perm1 = jax.lax.top_k(score, n2)
    xp = x1[perm1] * vals[:, None]
    ar2 = jnp.arange(n2, dtype=jnp.int32)
    perm1p = jnp.concatenate(
        [perm1, jnp.full((P2 - n2,), n1, jnp.int32)])
    rows = a1[perm1p, :].at[ar2, perm1].add(1.0)
    colst = a1t[perm1p, :].at[ar2, perm1].add(1.0)
    a2 = _mmt_mask(rows.astype(jnp.bfloat16), colst.astype(jnp.bfloat16))
    deg2 = jnp.sum(a2, axis=1)[:n2] + 2.0

    x2 = jax.nn.relu(_gcn(xp @ W1, a2, deg2, b1))

    score2 = jnp.tanh((x2 @ p2) / jnp.linalg.norm(p2))
    vals2, perm2 = jax.lax.top_k(score2, n3)
    xp2 = x2[perm2] * vals2[:, None]
    ar3 = jnp.arange(n3, dtype=jnp.int32)
    a2d = a2[:n2, :n2]
    rows2 = (jnp.pad(a2d[perm2, :], ((0, P3 - n3), (0, P2 - n2)))
             .at[ar3, perm2].add(1.0))
    cols2 = (jnp.pad(a2d[:, perm2], ((0, P2 - n2), (0, P3 - n3)))
             .at[perm2, ar3].add(1.0))
    a3 = _mm(rows2.astype(jnp.bfloat16), cols2.astype(jnp.bfloat16),
             mask_diag=True)
    deg3 = jnp.sum(a3, axis=1)[:n3] + 2.0

    x3 = jax.nn.relu(_gcn(xp2 @ W2, a3, deg3, b2))

    up = jnp.zeros_like(x2).at[perm2].set(x3)
    xu = jax.nn.relu(_gcn(jnp.concatenate([x2, up], axis=1) @ Wu0,
                          a2, deg2, bu0))
    up1 = jnp.zeros_like(x1).at[perm1].set(xu)
    out = _gcn(jnp.concatenate([x1, up1], axis=1) @ Wu1, a1, deg1, bu1)

    return pl.pallas_call(
        _logsoftmax_kernel,
        out_shape=jax.ShapeDtypeStruct(out.shape, out.dtype),
    )(out)
